# baseline (speedup 1.0000x reference)
import functools

import jax
import jax.numpy as jnp
from jax import lax
from jax.experimental import pallas as pl
from jax.experimental.pallas import tpu as pltpu

_BN_EPS = 1e-5
_VMEM_LIMIT = 48 * 1024 * 1024


def _stats_kernel(x_ref, g_ref, s_ref, xd_ref, *, cin, hw):
    i = pl.program_id(1)

    @pl.when(i == 0)
    def _init():
        g_ref[...] = jnp.zeros_like(g_ref)
        s_ref[...] = jnp.zeros_like(s_ref)

    x = x_ref[0]
    xr = jnp.reshape(x, (cin, hw))
    xb = xr.astype(jnp.bfloat16)
    xd_ref[0] = xb
    g_ref[0] += lax.dot_general(xb, xb, (((1,), (1,)), ((), ())),
                                preferred_element_type=jnp.float32)
    s_ref[0] += jnp.sum(xr, axis=1, keepdims=True)


def _apply_kernel(xd_ref, w_ref, b_ref, o_ref, *, cout, h, w):
    y = jnp.dot(w_ref[...], xd_ref[0], preferred_element_type=jnp.float32)
    o_ref[0] = jnp.reshape(y + b_ref[...], (cout, h, w))


@jax.jit
def _linear_block(x_nchw, conv_w, bn_gamma, bn_beta):
    N, Cin, H, W = x_nchw.shape
    Cout = conv_w.shape[0]
    HW = H * W
    M = N * HW
    inv_m = 1.0 / float(M)

    w2 = conv_w.reshape(Cout, Cin)

    ncore = 2 if N % 2 == 0 else 1
    per = N // ncore

    g_part, s_part, xd = pl.pallas_call(
        functools.partial(_stats_kernel, cin=Cin, hw=HW),
        out_shape=(jax.ShapeDtypeStruct((ncore, Cin, Cin), jnp.float32),
                   jax.ShapeDtypeStruct((ncore, Cin, 1), jnp.float32),
                   jax.ShapeDtypeStruct((N, Cin, HW), jnp.bfloat16)),
        grid=(ncore, per),
        in_specs=[pl.BlockSpec((1, Cin, H, W),
                               lambda c, i: (c * per + i, 0, 0, 0))],
        out_specs=(pl.BlockSpec((1, Cin, Cin), lambda c, i: (c, 0, 0)),
                   pl.BlockSpec((1, Cin, 1), lambda c, i: (c, 0, 0)),
                   pl.BlockSpec((1, Cin, HW), lambda c, i: (c * per + i, 0, 0))),
        compiler_params=pltpu.CompilerParams(
            dimension_semantics=("parallel", "arbitrary"),
            vmem_limit_bytes=_VMEM_LIMIT,
        ),
        cost_estimate=pl.CostEstimate(
            flops=int(2 * M * Cin * Cin + M * Cin),
            transcendentals=0,
            bytes_accessed=int(4 * N * Cin * H * 128 + 2 * N * Cin * HW),
        ),
    )(x_nchw)

    G = jnp.sum(g_part, axis=0)
    s = jnp.sum(s_part, axis=0)[:, 0]
    mean = (w2 @ s) * inv_m
    ey2 = jnp.sum((w2 @ G) * w2, axis=1) * inv_m
    var = jnp.maximum(ey2 - mean * mean, 0.0)
    inv_std = lax.rsqrt(var + _BN_EPS)
    scale = bn_gamma * inv_std
    shift = (bn_beta - mean * scale).reshape(Cout, 1)
    w_folded = (w2 * scale[:, None]).astype(jnp.bfloat16)

    out = pl.pallas_call(
        functools.partial(_apply_kernel, cout=Cout, h=H, w=W),
        out_shape=jax.ShapeDtypeStruct((N, Cout, H, W), jnp.float32),
        grid=(N,),
        in_specs=[
            pl.BlockSpec((1, Cin, HW), lambda n: (n, 0, 0)),
            pl.BlockSpec((Cout, Cin), lambda n: (0, 0)),
            pl.BlockSpec((Cout, 1), lambda n: (0, 0)),
        ],
        out_specs=pl.BlockSpec((1, Cout, H, W), lambda n: (n, 0, 0, 0)),
        compiler_params=pltpu.CompilerParams(
            dimension_semantics=("parallel",),
            vmem_limit_bytes=_VMEM_LIMIT,
        ),
        cost_estimate=pl.CostEstimate(
            flops=int(2 * M * Cin * Cout + M * Cout),
            transcendentals=0,
            bytes_accessed=int(2 * N * Cin * HW + 4 * N * Cout * H * 128),
        ),
    )(xd, w_folded, shift)

    return out


def kernel(x_nchw, conv_w, bn_gamma, bn_beta):
    return _linear_block(x_nchw, conv_w, bn_gamma, bn_beta)


# speedup vs baseline: 4.5216x; 4.5216x over previous
import functools

import jax
import jax.numpy as jnp
from jax import lax
from jax.experimental import pallas as pl
from jax.experimental.pallas import tpu as pltpu

_BN_EPS = 1e-5
_VMEM_LIMIT = 48 * 1024 * 1024


def _stats_kernel(x_ref, g_ref, s_ref, *, cin, hw):
    i = pl.program_id(1)

    @pl.when(i == 0)
    def _init():
        g_ref[...] = jnp.zeros_like(g_ref)
        s_ref[...] = jnp.zeros_like(s_ref)

    x = jnp.reshape(x_ref[0], (hw, cin))
    xb = x.astype(jnp.bfloat16)
    g_ref[0] += lax.dot_general(xb, xb, (((0,), (0,)), ((), ())),
                                preferred_element_type=jnp.float32)
    s_ref[0] += jnp.sum(x, axis=0, keepdims=True)


def _apply_kernel(x_ref, w_ref, b_ref, o_ref, *, h, w, cout, hw, cin):
    xb = jnp.reshape(x_ref[0], (hw, cin)).astype(jnp.bfloat16)
    y = jnp.dot(xb, w_ref[...], preferred_element_type=jnp.float32)
    o_ref[0] = jnp.reshape(y + b_ref[...], (h, w, cout))


@jax.jit
def _linear_block(x_nchw, conv_w, bn_gamma, bn_beta):
    N, Cin, H, W = x_nchw.shape
    Cout = conv_w.shape[0]
    HW = H * W
    M = N * HW
    inv_m = 1.0 / float(M)

    xt = jnp.transpose(x_nchw, (0, 2, 3, 1))
    w2 = conv_w.reshape(Cout, Cin)

    ncore = 2 if N % 2 == 0 else 1
    per = N // ncore

    g_part, s_part = pl.pallas_call(
        functools.partial(_stats_kernel, cin=Cin, hw=HW),
        out_shape=(jax.ShapeDtypeStruct((ncore, Cin, Cin), jnp.float32),
                   jax.ShapeDtypeStruct((ncore, 1, Cin), jnp.float32)),
        grid=(ncore, per),
        in_specs=[pl.BlockSpec((1, H, W, Cin),
                               lambda c, i: (c * per + i, 0, 0, 0))],
        out_specs=(pl.BlockSpec((1, Cin, Cin), lambda c, i: (c, 0, 0)),
                   pl.BlockSpec((1, 1, Cin), lambda c, i: (c, 0, 0))),
        compiler_params=pltpu.CompilerParams(
            dimension_semantics=("parallel", "arbitrary"),
            vmem_limit_bytes=_VMEM_LIMIT,
        ),
        cost_estimate=pl.CostEstimate(
            flops=int(2 * M * Cin * Cin + M * Cin),
            transcendentals=0,
            bytes_accessed=int(4 * (N * Cin * HW + ncore * Cin * (Cin + 1))),
        ),
    )(xt)

    G = jnp.sum(g_part, axis=0)
    s = jnp.sum(s_part, axis=0)[0]
    mean = (w2 @ s) * inv_m
    ey2 = jnp.sum((w2 @ G) * w2, axis=1) * inv_m
    var = jnp.maximum(ey2 - mean * mean, 0.0)
    inv_std = lax.rsqrt(var + _BN_EPS)
    scale = bn_gamma * inv_std
    shift = (bn_beta - mean * scale).reshape(1, Cout)
    w_folded = (w2 * scale[:, None]).T.astype(jnp.bfloat16)

    out_nhwc = pl.pallas_call(
        functools.partial(_apply_kernel, h=H, w=W, cout=Cout, hw=HW, cin=Cin),
        out_shape=jax.ShapeDtypeStruct((N, H, W, Cout), jnp.float32),
        grid=(N,),
        in_specs=[
            pl.BlockSpec((1, H, W, Cin), lambda n: (n, 0, 0, 0)),
            pl.BlockSpec((Cin, Cout), lambda n: (0, 0)),
            pl.BlockSpec((1, Cout), lambda n: (0, 0)),
        ],
        out_specs=pl.BlockSpec((1, H, W, Cout), lambda n: (n, 0, 0, 0)),
        compiler_params=pltpu.CompilerParams(
            dimension_semantics=("parallel",),
            vmem_limit_bytes=_VMEM_LIMIT,
        ),
        cost_estimate=pl.CostEstimate(
            flops=int(2 * M * Cin * Cout + M * Cout),
            transcendentals=0,
            bytes_accessed=int(4 * (N * (Cin + Cout) * HW + Cout * (Cin + 1))),
        ),
    )(xt, w_folded, shift)

    return jnp.transpose(out_nhwc, (0, 3, 1, 2))

def kernel(x_nchw, conv_w, bn_gamma, bn_beta):
    return _linear_block(x_nchw, conv_w, bn_gamma, bn_beta)


# speedup vs baseline: 5.0329x; 1.1131x over previous
import functools

import jax
import jax.numpy as jnp
from jax import lax
from jax.experimental import pallas as pl
from jax.experimental.pallas import tpu as pltpu

_BN_EPS = 1e-5
_VMEM_LIMIT = 48 * 1024 * 1024


def _stats_kernel(x_ref, g_ref, s_ref, *, cin, rows):
    i = pl.program_id(1)

    @pl.when(i == 0)
    def _init():
        g_ref[...] = jnp.zeros_like(g_ref)
        s_ref[...] = jnp.zeros_like(s_ref)

    x = jnp.reshape(x_ref[...], (rows, cin))
    xb = x.astype(jnp.bfloat16)
    g_ref[0] += lax.dot_general(xb, xb, (((0,), (0,)), ((), ())),
                                preferred_element_type=jnp.float32)
    s_ref[0] += jnp.sum(x, axis=0, keepdims=True)


def _apply_kernel(x_ref, w_ref, b_ref, o_ref, *, b, h, w, cout, rows, cin):
    xb = jnp.reshape(x_ref[...], (rows, cin)).astype(jnp.bfloat16)
    y = jnp.dot(xb, w_ref[...], preferred_element_type=jnp.float32)
    o_ref[...] = jnp.reshape(y + b_ref[...], (b, h, w, cout))


@jax.jit
def _linear_block(x_nchw, conv_w, bn_gamma, bn_beta):
    N, Cin, H, W = x_nchw.shape
    Cout = conv_w.shape[0]
    HW = H * W
    M = N * HW
    inv_m = 1.0 / float(M)

    xt = jnp.transpose(x_nchw, (0, 2, 3, 1))
    w2 = conv_w.reshape(Cout, Cin)

    ncore = 2 if N % 2 == 0 else 1
    nb = 2 if (N // ncore) % 2 == 0 else 1
    per = N // (ncore * nb)

    g_part, s_part = pl.pallas_call(
        functools.partial(_stats_kernel, cin=Cin, rows=nb * HW),
        out_shape=(jax.ShapeDtypeStruct((ncore, Cin, Cin), jnp.float32),
                   jax.ShapeDtypeStruct((ncore, 1, Cin), jnp.float32)),
        grid=(ncore, per),
        in_specs=[pl.BlockSpec((nb, H, W, Cin),
                               lambda c, i: (c * per + i, 0, 0, 0))],
        out_specs=(pl.BlockSpec((1, Cin, Cin), lambda c, i: (c, 0, 0)),
                   pl.BlockSpec((1, 1, Cin), lambda c, i: (c, 0, 0))),
        compiler_params=pltpu.CompilerParams(
            dimension_semantics=("parallel", "arbitrary"),
            vmem_limit_bytes=_VMEM_LIMIT,
        ),
        cost_estimate=pl.CostEstimate(
            flops=int(2 * M * Cin * Cin + M * Cin),
            transcendentals=0,
            bytes_accessed=int(4 * (N * Cin * HW + ncore * Cin * (Cin + 1))),
        ),
    )(xt)

    G = jnp.sum(g_part, axis=0)
    s = jnp.sum(s_part, axis=0)[0]
    mean = (w2 @ s) * inv_m
    ey2 = jnp.sum((w2 @ G) * w2, axis=1) * inv_m
    var = jnp.maximum(ey2 - mean * mean, 0.0)
    inv_std = lax.rsqrt(var + _BN_EPS)
    scale = bn_gamma * inv_std
    shift = (bn_beta - mean * scale).reshape(1, Cout)
    w_folded = (w2 * scale[:, None]).T.astype(jnp.bfloat16)

    out_nhwc = pl.pallas_call(
        functools.partial(_apply_kernel, b=nb, h=H, w=W, cout=Cout,
                          rows=nb * HW, cin=Cin),
        out_shape=jax.ShapeDtypeStruct((N, H, W, Cout), jnp.float32),
        grid=(N // nb,),
        in_specs=[
            pl.BlockSpec((nb, H, W, Cin), lambda n: (n, 0, 0, 0)),
            pl.BlockSpec((Cin, Cout), lambda n: (0, 0)),
            pl.BlockSpec((1, Cout), lambda n: (0, 0)),
        ],
        out_specs=pl.BlockSpec((nb, H, W, Cout), lambda n: (n, 0, 0, 0)),
        compiler_params=pltpu.CompilerParams(
            dimension_semantics=("parallel",),
            vmem_limit_bytes=_VMEM_LIMIT,
        ),
        cost_estimate=pl.CostEstimate(
            flops=int(2 * M * Cin * Cout + M * Cout),
            transcendentals=0,
            bytes_accessed=int(4 * (N * (Cin + Cout) * HW + Cout * (Cin + 1))),
        ),
    )(xt, w_folded, shift)

    return jnp.transpose(out_nhwc, (0, 3, 1, 2))

def kernel(x_nchw, conv_w, bn_gamma, bn_beta):
    return _linear_block(x_nchw, conv_w, bn_gamma, bn_beta)


# speedup vs baseline: 5.0570x; 1.0048x over previous
import functools

import jax
import jax.numpy as jnp
from jax import lax
from jax.experimental import pallas as pl
from jax.experimental.pallas import tpu as pltpu

_BN_EPS = 1e-5
_VMEM_LIMIT = 48 * 1024 * 1024


def _stats_kernel(x_ref, g_ref, s_ref, *, cin, rows):
    i = pl.program_id(1)

    @pl.when(i == 0)
    def _init():
        g_ref[...] = jnp.zeros_like(g_ref)
        s_ref[...] = jnp.zeros_like(s_ref)

    x = jnp.reshape(x_ref[...], (rows, cin))
    xb = x.astype(jnp.bfloat16)
    g_ref[0] += lax.dot_general(xb, xb, (((0,), (0,)), ((), ())),
                                preferred_element_type=jnp.float32)
    s_ref[0] += jnp.sum(x, axis=0, keepdims=True)


def _apply_kernel(x_ref, w_ref, b_ref, o_ref, *, b, h, w, cout, rows, cin):
    xb = jnp.reshape(x_ref[...], (rows, cin)).astype(jnp.bfloat16)
    y = jnp.dot(xb, w_ref[...], preferred_element_type=jnp.float32)
    o_ref[...] = jnp.reshape(y + b_ref[...], (b, h, w, cout))


@jax.jit
def _linear_block(x_nchw, conv_w, bn_gamma, bn_beta):
    N, Cin, H, W = x_nchw.shape
    Cout = conv_w.shape[0]
    HW = H * W
    M = N * HW
    inv_m = 1.0 / float(M)

    xt = jnp.transpose(x_nchw, (0, 2, 3, 1))
    w2 = conv_w.reshape(Cout, Cin)

    ncore = 2 if N % 2 == 0 else 1
    nb = 2 if (N // ncore) % 2 == 0 else 1
    nb1 = 4 if (N // ncore) % 4 == 0 else nb
    per = N // (ncore * nb1)

    g_part, s_part = pl.pallas_call(
        functools.partial(_stats_kernel, cin=Cin, rows=nb1 * HW),
        out_shape=(jax.ShapeDtypeStruct((ncore, Cin, Cin), jnp.float32),
                   jax.ShapeDtypeStruct((ncore, 1, Cin), jnp.float32)),
        grid=(ncore, per),
        in_specs=[pl.BlockSpec((nb1, H, W, Cin),
                               lambda c, i: (c * per + i, 0, 0, 0))],
        out_specs=(pl.BlockSpec((1, Cin, Cin), lambda c, i: (c, 0, 0)),
                   pl.BlockSpec((1, 1, Cin), lambda c, i: (c, 0, 0))),
        compiler_params=pltpu.CompilerParams(
            dimension_semantics=("parallel", "arbitrary"),
            vmem_limit_bytes=_VMEM_LIMIT,
        ),
        cost_estimate=pl.CostEstimate(
            flops=int(2 * M * Cin * Cin + M * Cin),
            transcendentals=0,
            bytes_accessed=int(4 * (N * Cin * HW + ncore * Cin * (Cin + 1))),
        ),
    )(xt)

    G = jnp.sum(g_part, axis=0)
    s = jnp.sum(s_part, axis=0)[0]
    mean = (w2 @ s) * inv_m
    ey2 = jnp.sum((w2 @ G) * w2, axis=1) * inv_m
    var = jnp.maximum(ey2 - mean * mean, 0.0)
    inv_std = lax.rsqrt(var + _BN_EPS)
    scale = bn_gamma * inv_std
    shift = (bn_beta - mean * scale).reshape(1, Cout)
    w_folded = (w2 * scale[:, None]).T.astype(jnp.bfloat16)

    out_nhwc = pl.pallas_call(
        functools.partial(_apply_kernel, b=nb, h=H, w=W, cout=Cout,
                          rows=nb * HW, cin=Cin),
        out_shape=jax.ShapeDtypeStruct((N, H, W, Cout), jnp.float32),
        grid=(N // nb,),
        in_specs=[
            pl.BlockSpec((nb, H, W, Cin), lambda n: (n, 0, 0, 0)),
            pl.BlockSpec((Cin, Cout), lambda n: (0, 0)),
            pl.BlockSpec((1, Cout), lambda n: (0, 0)),
        ],
        out_specs=pl.BlockSpec((nb, H, W, Cout), lambda n: (n, 0, 0, 0)),
        compiler_params=pltpu.CompilerParams(
            dimension_semantics=("parallel",),
            vmem_limit_bytes=_VMEM_LIMIT,
        ),
        cost_estimate=pl.CostEstimate(
            flops=int(2 * M * Cin * Cout + M * Cout),
            transcendentals=0,
            bytes_accessed=int(4 * (N * (Cin + Cout) * HW + Cout * (Cin + 1))),
        ),
    )(xt, w_folded, shift)

    return jnp.transpose(out_nhwc, (0, 3, 1, 2))

def kernel(x_nchw, conv_w, bn_gamma, bn_beta):
    return _linear_block(x_nchw, conv_w, bn_gamma, bn_beta)


# speedup vs baseline: 5.0607x; 1.0007x over previous
import functools

import jax
import jax.numpy as jnp
from jax import lax
from jax.experimental import pallas as pl
from jax.experimental.pallas import tpu as pltpu

_BN_EPS = 1e-5
_VMEM_LIMIT = 48 * 1024 * 1024


def _stats_kernel(xa_ref, xb_ref, g_ref, s_ref, *, cin, rows):
    i = pl.program_id(1)

    @pl.when(i == 0)
    def _init():
        g_ref[...] = jnp.zeros_like(g_ref)
        s_ref[...] = jnp.zeros_like(s_ref)

    xa = jnp.reshape(xa_ref[...], (rows, cin))
    xb = jnp.reshape(xb_ref[...], (rows, cin))
    xab = xa.astype(jnp.bfloat16)
    xbb = xb.astype(jnp.bfloat16)
    g = lax.dot_general(xab, xab, (((0,), (0,)), ((), ())),
                        preferred_element_type=jnp.float32)
    g += lax.dot_general(xbb, xbb, (((0,), (0,)), ((), ())),
                         preferred_element_type=jnp.float32)
    g_ref[0] += g
    s_ref[0] += (jnp.sum(xa, axis=0, keepdims=True) +
                 jnp.sum(xb, axis=0, keepdims=True))


def _apply_kernel(x_ref, w_ref, b_ref, o_ref, *, b, h, w, cout, rows, cin):
    xb = jnp.reshape(x_ref[...], (rows, cin)).astype(jnp.bfloat16)
    y = jnp.dot(xb, w_ref[...], preferred_element_type=jnp.float32)
    o_ref[...] = jnp.reshape(y + b_ref[...], (b, h, w, cout))


@jax.jit
def _linear_block(x_nchw, conv_w, bn_gamma, bn_beta):
    N, Cin, H, W = x_nchw.shape
    Cout = conv_w.shape[0]
    HW = H * W
    M = N * HW
    inv_m = 1.0 / float(M)

    xt = jnp.transpose(x_nchw, (0, 2, 3, 1))
    w2 = conv_w.reshape(Cout, Cin)

    ncore = 2 if N % 2 == 0 else 1
    nb = 2 if (N // ncore) % 2 == 0 else 1
    nb1 = 4 if (N // ncore) % 4 == 0 else nb
    per = N // (ncore * nb1)

    g_part, s_part = pl.pallas_call(
        functools.partial(_stats_kernel, cin=Cin, rows=nb1 * (H // 2) * W),
        out_shape=(jax.ShapeDtypeStruct((ncore, Cin, Cin), jnp.float32),
                   jax.ShapeDtypeStruct((ncore, 1, Cin), jnp.float32)),
        grid=(ncore, per),
        in_specs=[pl.BlockSpec((nb1, H // 2, W, Cin),
                               lambda c, i: (c * per + i, 0, 0, 0)),
                  pl.BlockSpec((nb1, H // 2, W, Cin),
                               lambda c, i: (c * per + i, 1, 0, 0))],
        out_specs=(pl.BlockSpec((1, Cin, Cin), lambda c, i: (c, 0, 0)),
                   pl.BlockSpec((1, 1, Cin), lambda c, i: (c, 0, 0))),
        compiler_params=pltpu.CompilerParams(
            dimension_semantics=("parallel", "arbitrary"),
            vmem_limit_bytes=_VMEM_LIMIT,
        ),
        cost_estimate=pl.CostEstimate(
            flops=int(2 * M * Cin * Cin + M * Cin),
            transcendentals=0,
            bytes_accessed=int(4 * (N * Cin * HW + ncore * Cin * (Cin + 1))),
        ),
    )(xt, xt)

    G = jnp.sum(g_part, axis=0)
    s = jnp.sum(s_part, axis=0)[0]
    mean = (w2 @ s) * inv_m
    ey2 = jnp.sum((w2 @ G) * w2, axis=1) * inv_m
    var = jnp.maximum(ey2 - mean * mean, 0.0)
    inv_std = lax.rsqrt(var + _BN_EPS)
    scale = bn_gamma * inv_std
    shift = (bn_beta - mean * scale).reshape(1, Cout)
    w_folded = (w2 * scale[:, None]).T.astype(jnp.bfloat16)

    out_nhwc = pl.pallas_call(
        functools.partial(_apply_kernel, b=nb, h=H, w=W, cout=Cout,
                          rows=nb * HW, cin=Cin),
        out_shape=jax.ShapeDtypeStruct((N, H, W, Cout), jnp.float32),
        grid=(N // nb,),
        in_specs=[
            pl.BlockSpec((nb, H, W, Cin), lambda n: (n, 0, 0, 0)),
            pl.BlockSpec((Cin, Cout), lambda n: (0, 0)),
            pl.BlockSpec((1, Cout), lambda n: (0, 0)),
        ],
        out_specs=pl.BlockSpec((nb, H, W, Cout), lambda n: (n, 0, 0, 0)),
        compiler_params=pltpu.CompilerParams(
            dimension_semantics=("parallel",),
            vmem_limit_bytes=_VMEM_LIMIT,
        ),
        cost_estimate=pl.CostEstimate(
            flops=int(2 * M * Cin * Cout + M * Cout),
            transcendentals=0,
            bytes_accessed=int(4 * (N * (Cin + Cout) * HW + Cout * (Cin + 1))),
        ),
    )(xt, w_folded, shift)

    return jnp.transpose(out_nhwc, (0, 3, 1, 2))

def kernel(x_nchw, conv_w, bn_gamma, bn_beta):
    return _linear_block(x_nchw, conv_w, bn_gamma, bn_beta)


# speedup vs baseline: 5.1266x; 1.0130x over previous
import functools

import jax
import jax.numpy as jnp
from jax import lax
from jax.experimental import pallas as pl
from jax.experimental.pallas import tpu as pltpu

_BN_EPS = 1e-5
_VMEM_LIMIT = 48 * 1024 * 1024


def _stats_kernel(x_ref, g_ref, s_ref, xd_ref, *, cin, rows):
    i = pl.program_id(1)

    @pl.when(i == 0)
    def _init():
        g_ref[...] = jnp.zeros_like(g_ref)
        s_ref[...] = jnp.zeros_like(s_ref)

    x = jnp.reshape(x_ref[...], (rows, cin))
    xb = x.astype(jnp.bfloat16)
    xd_ref[...] = jnp.reshape(xb, xd_ref.shape)
    g_ref[0] += lax.dot_general(xb, xb, (((0,), (0,)), ((), ())),
                                preferred_element_type=jnp.float32)
    s_ref[0] += jnp.sum(x, axis=0, keepdims=True)


def _apply_kernel(xd_ref, w_ref, b_ref, o_ref, *, b, h, w, cout, rows, cin):
    xb = jnp.reshape(xd_ref[...], (rows, cin))
    y = jnp.dot(xb, w_ref[...], preferred_element_type=jnp.float32)
    o_ref[...] = jnp.reshape(y + b_ref[...], (b, h, w, cout))


@jax.jit
def _linear_block(x_nchw, conv_w, bn_gamma, bn_beta):
    N, Cin, H, W = x_nchw.shape
    Cout = conv_w.shape[0]
    HW = H * W
    M = N * HW
    inv_m = 1.0 / float(M)

    xt = jnp.transpose(x_nchw, (0, 2, 3, 1))
    w2 = conv_w.reshape(Cout, Cin)

    ncore = 2 if N % 2 == 0 else 1
    nb1 = 2 if (N // ncore) % 2 == 0 else 1
    nb2 = 4 if (N // ncore) % 4 == 0 else 1
    per = N // (ncore * nb1)

    g_part, s_part, xd = pl.pallas_call(
        functools.partial(_stats_kernel, cin=Cin, rows=nb1 * HW),
        out_shape=(jax.ShapeDtypeStruct((ncore, Cin, Cin), jnp.float32),
                   jax.ShapeDtypeStruct((ncore, 1, Cin), jnp.float32),
                   jax.ShapeDtypeStruct((N, HW, Cin), jnp.bfloat16)),
        grid=(ncore, per),
        in_specs=[pl.BlockSpec((nb1, H, W, Cin),
                               lambda c, i: (c * per + i, 0, 0, 0))],
        out_specs=(pl.BlockSpec((1, Cin, Cin), lambda c, i: (c, 0, 0)),
                   pl.BlockSpec((1, 1, Cin), lambda c, i: (c, 0, 0)),
                   pl.BlockSpec((nb1, HW, Cin),
                                lambda c, i: (c * per + i, 0, 0))),
        compiler_params=pltpu.CompilerParams(
            dimension_semantics=("parallel", "arbitrary"),
            vmem_limit_bytes=_VMEM_LIMIT,
        ),
        cost_estimate=pl.CostEstimate(
            flops=int(2 * M * Cin * Cin + M * Cin),
            transcendentals=0,
            bytes_accessed=int(4 * N * Cin * HW + 2 * N * Cin * HW),
        ),
    )(xt)

    G = jnp.sum(g_part, axis=0)
    s = jnp.sum(s_part, axis=0)[0]
    mean = (w2 @ s) * inv_m
    ey2 = jnp.sum((w2 @ G) * w2, axis=1) * inv_m
    var = jnp.maximum(ey2 - mean * mean, 0.0)
    inv_std = lax.rsqrt(var + _BN_EPS)
    scale = bn_gamma * inv_std
    shift = (bn_beta - mean * scale).reshape(1, Cout)
    w_folded = (w2 * scale[:, None]).T.astype(jnp.bfloat16)

    out_nhwc = pl.pallas_call(
        functools.partial(_apply_kernel, b=nb2, h=H, w=W, cout=Cout,
                          rows=nb2 * HW, cin=Cin),
        out_shape=jax.ShapeDtypeStruct((N, H, W, Cout), jnp.float32),
        grid=(N // nb2,),
        in_specs=[
            pl.BlockSpec((nb2, HW, Cin), lambda n: (n, 0, 0)),
            pl.BlockSpec((Cin, Cout), lambda n: (0, 0)),
            pl.BlockSpec((1, Cout), lambda n: (0, 0)),
        ],
        out_specs=pl.BlockSpec((nb2, H, W, Cout), lambda n: (n, 0, 0, 0)),
        compiler_params=pltpu.CompilerParams(
            dimension_semantics=("parallel",),
            vmem_limit_bytes=_VMEM_LIMIT,
        ),
        cost_estimate=pl.CostEstimate(
            flops=int(2 * M * Cin * Cout + M * Cout),
            transcendentals=0,
            bytes_accessed=int(2 * N * Cin * HW + 4 * N * Cout * HW),
        ),
    )(xd, w_folded, shift)

    return jnp.transpose(out_nhwc, (0, 3, 1, 2))

def kernel(x_nchw, conv_w, bn_gamma, bn_beta):
    return _linear_block(x_nchw, conv_w, bn_gamma, bn_beta)


# speedup vs baseline: 5.3687x; 1.0472x over previous
import functools

import jax
import jax.numpy as jnp
from jax import lax
from jax.experimental import pallas as pl
from jax.experimental.pallas import tpu as pltpu

_BN_EPS = 1e-5
_VMEM_LIMIT = 48 * 1024 * 1024


def _stats_kernel(x_ref, g_ref, s_ref, xd_ref, *, cin, rows):
    i = pl.program_id(1)

    @pl.when(i == 0)
    def _init():
        g_ref[...] = jnp.zeros_like(g_ref)
        s_ref[...] = jnp.zeros_like(s_ref)

    x = jnp.reshape(x_ref[...], (rows, cin))
    xb = x.astype(jnp.bfloat16)
    xd_ref[...] = jnp.reshape(xb, xd_ref.shape)
    g_ref[0] += lax.dot_general(xb, xb, (((0,), (0,)), ((), ())),
                                preferred_element_type=jnp.float32)
    s_ref[0] += jnp.sum(x, axis=0, keepdims=True)


def _apply_kernel(xd_ref, w_ref, b_ref, o_ref, *, b, h, w, cout, rows, cin):
    xb = jnp.reshape(xd_ref[...], (rows, cin))
    y = lax.dot_general(xb, w_ref[...], (((1,), (1,)), ((), ())),
                        preferred_element_type=jnp.float32)
    o_ref[...] = jnp.reshape(y + b_ref[...], (b, h, w, cout))


@jax.jit
def _linear_block(x_nchw, conv_w, bn_gamma, bn_beta):
    N, Cin, H, W = x_nchw.shape
    Cout = conv_w.shape[0]
    HW = H * W
    M = N * HW
    inv_m = 1.0 / float(M)

    xt = jnp.transpose(x_nchw, (0, 2, 3, 1))
    w2 = conv_w.reshape(Cout, Cin)

    ncore = 2 if N % 2 == 0 else 1
    nb1 = 4 if (N // ncore) % 4 == 0 else 1
    nb2 = 4 if (N // ncore) % 4 == 0 else 1
    per = N // (ncore * nb1)

    g_part, s_part, xd = pl.pallas_call(
        functools.partial(_stats_kernel, cin=Cin, rows=nb1 * HW),
        out_shape=(jax.ShapeDtypeStruct((ncore, Cin, Cin), jnp.float32),
                   jax.ShapeDtypeStruct((ncore, 1, Cin), jnp.float32),
                   jax.ShapeDtypeStruct((N, HW, Cin), jnp.bfloat16)),
        grid=(ncore, per),
        in_specs=[pl.BlockSpec((nb1, H, W, Cin),
                               lambda c, i: (c * per + i, 0, 0, 0))],
        out_specs=(pl.BlockSpec((1, Cin, Cin), lambda c, i: (c, 0, 0)),
                   pl.BlockSpec((1, 1, Cin), lambda c, i: (c, 0, 0)),
                   pl.BlockSpec((nb1, HW, Cin),
                                lambda c, i: (c * per + i, 0, 0))),
        compiler_params=pltpu.CompilerParams(
            dimension_semantics=("parallel", "arbitrary"),
            vmem_limit_bytes=_VMEM_LIMIT,
        ),
        cost_estimate=pl.CostEstimate(
            flops=int(2 * M * Cin * Cin + M * Cin),
            transcendentals=0,
            bytes_accessed=int(4 * N * Cin * HW + 2 * N * Cin * HW),
        ),
    )(xt)

    G = jnp.sum(g_part, axis=0)
    s = jnp.sum(s_part, axis=0)[0]
    mean = (w2 @ s) * inv_m
    ey2 = jnp.sum((w2 @ G) * w2, axis=1) * inv_m
    var = jnp.maximum(ey2 - mean * mean, 0.0)
    inv_std = lax.rsqrt(var + _BN_EPS)
    scale = bn_gamma * inv_std
    shift = (bn_beta - mean * scale).reshape(1, Cout)
    w_folded = (w2 * scale[:, None]).astype(jnp.bfloat16)

    out_nhwc = pl.pallas_call(
        functools.partial(_apply_kernel, b=nb2, h=H, w=W, cout=Cout,
                          rows=nb2 * HW, cin=Cin),
        out_shape=jax.ShapeDtypeStruct((N, H, W, Cout), jnp.float32),
        grid=(N // nb2,),
        in_specs=[
            pl.BlockSpec((nb2, HW, Cin), lambda n: (n, 0, 0)),
            pl.BlockSpec((Cout, Cin), lambda n: (0, 0)),
            pl.BlockSpec((1, Cout), lambda n: (0, 0)),
        ],
        out_specs=pl.BlockSpec((nb2, H, W, Cout), lambda n: (n, 0, 0, 0)),
        compiler_params=pltpu.CompilerParams(
            dimension_semantics=("parallel",),
            vmem_limit_bytes=_VMEM_LIMIT,
        ),
        cost_estimate=pl.CostEstimate(
            flops=int(2 * M * Cin * Cout + M * Cout),
            transcendentals=0,
            bytes_accessed=int(2 * N * Cin * HW + 4 * N * Cout * HW),
        ),
    )(xd, w_folded, shift)

    return jnp.transpose(out_nhwc, (0, 3, 1, 2))

def kernel(x_nchw, conv_w, bn_gamma, bn_beta):
    return _linear_block(x_nchw, conv_w, bn_gamma, bn_beta)
